# Initial kernel scaffold; baseline (speedup 1.0000x reference)
import jax
import jax.numpy as jnp
from jax.experimental import pallas as pl


def kernel(piece_positions, king_positions, input_weights, bias):
    raise NotImplementedError("write your pallas kernel here")


# speedup vs baseline: 21.0734x; 21.0734x over previous
import jax
import jax.numpy as jnp
from jax.experimental import pallas as pl
from jax.experimental.pallas import tpu as pltpu


def _halfkp_kernel(kings_ref, p_ref, w_ref, bias_ref, out_ref):
    k = pl.program_id(0)

    eq = (kings_ref[...] == k).astype(jnp.float32)
    coeff = eq[:, 0:1] + eq[:, 1:2]

    q = coeff.astype(jnp.bfloat16) * p_ref[...]
    w = w_ref[0, :640, :].astype(jnp.bfloat16)
    acc = jnp.dot(q, w, preferred_element_type=jnp.float32)
    acc += coeff * w_ref[0, 640:641, :]

    @pl.when(k == 0)
    def _init():
        out_ref[...] = jnp.broadcast_to(bias_ref[...], out_ref.shape)

    out_ref[...] += acc


def kernel(piece_positions, king_positions, input_weights, bias):
    b = piece_positions.shape[0]
    n_kings, n_rows, n_out = input_weights.shape
    n_feat = n_rows - 1

    p = piece_positions.reshape(b, n_feat).astype(jnp.bfloat16)
    kings = king_positions.astype(jnp.int32)
    bias2 = bias.reshape(1, n_out)

    return pl.pallas_call(
        _halfkp_kernel,
        grid=(n_kings,),
        in_specs=[
            pl.BlockSpec((b, 2), lambda k: (0, 0)),
            pl.BlockSpec((b, n_feat), lambda k: (0, 0)),
            pl.BlockSpec((1, n_rows, n_out), lambda k: (k, 0, 0)),
            pl.BlockSpec((1, n_out), lambda k: (0, 0)),
        ],
        out_specs=pl.BlockSpec((b, n_out), lambda k: (0, 0)),
        out_shape=jax.ShapeDtypeStruct((b, n_out), jnp.float32),
        compiler_params=pltpu.CompilerParams(
            dimension_semantics=("arbitrary",),
        ),
    )(kings, p, input_weights, bias2)


# speedup vs baseline: 21.6532x; 1.0275x over previous
import jax
import jax.numpy as jnp
from jax.experimental import pallas as pl
from jax.experimental.pallas import tpu as pltpu


def _halfkp_kernel(kings_ref, p_ref, w_ref, bias_ref, out_ref):
    k = pl.program_id(0)

    eq = (kings_ref[...] == k).astype(jnp.float32)
    coeff = eq[:, 0:1] + eq[:, 1:2]

    w = w_ref[0, :640, :].astype(jnp.bfloat16)
    mm = jnp.dot(p_ref[...], w, preferred_element_type=jnp.float32)
    acc = coeff * (mm + w_ref[0, 640:641, :])

    @pl.when(k == 0)
    def _init():
        out_ref[...] = jnp.broadcast_to(bias_ref[...], out_ref.shape)

    out_ref[...] += acc


def kernel(piece_positions, king_positions, input_weights, bias):
    b = piece_positions.shape[0]
    n_kings, n_rows, n_out = input_weights.shape
    n_feat = n_rows - 1

    p = piece_positions.reshape(b, n_feat).astype(jnp.bfloat16)
    kings = king_positions.astype(jnp.int32)
    bias2 = bias.reshape(1, n_out)

    return pl.pallas_call(
        _halfkp_kernel,
        grid=(n_kings,),
        in_specs=[
            pl.BlockSpec((b, 2), lambda k: (0, 0)),
            pl.BlockSpec((b, n_feat), lambda k: (0, 0)),
            pl.BlockSpec((1, n_rows, n_out), lambda k: (k, 0, 0)),
            pl.BlockSpec((1, n_out), lambda k: (0, 0)),
        ],
        out_specs=pl.BlockSpec((b, n_out), lambda k: (0, 0)),
        out_shape=jax.ShapeDtypeStruct((b, n_out), jnp.float32),
        compiler_params=pltpu.CompilerParams(
            dimension_semantics=("arbitrary",),
        ),
    )(kings, p, input_weights, bias2)
